# baseline (speedup 1.0000x reference)
import functools

import jax
import jax.numpy as jnp
from jax import lax
from jax.experimental import pallas as pl
from jax.experimental.pallas import tpu as pltpu
from jax.experimental.pallas import tpu_sc as plsc

_NS = 16


@functools.lru_cache(maxsize=None)
def _build(B, S, D):
    assert B % _NS == 0
    rows_per_w = B // _NS
    mesh = plsc.VectorSubcoreMesh(
        core_axis_name="c", subcore_axis_name="s", num_cores=1)

    @functools.partial(
        pl.kernel,
        mesh=mesh,
        out_type=jax.ShapeDtypeStruct((B, D), jnp.float32),
        scratch_types=[
            pltpu.VMEM((16,), jnp.int32),
            pltpu.SemaphoreType.DMA,
            pltpu.SemaphoreType.DMA,
        ],
    )
    def gather_kernel(flat_hbm, idx_hbm, out_hbm, ids_v, idx_sem, sem):
        wid = lax.axis_index("s")
        base = wid * rows_per_w
        idx_cp = pltpu.make_async_copy(
            idx_hbm.at[pl.ds(base, rows_per_w)],
            ids_v.at[pl.ds(0, rows_per_w)],
            idx_sem,
        )
        idx_cp.start()
        idx_cp.wait()
        ids = ids_v[...]
        for j in range(rows_per_w):
            gid = (base + j) * S + ids[j]
            pltpu.make_async_copy(
                flat_hbm.at[pl.ds(gid, 1)],
                out_hbm.at[pl.ds(base + j, 1)],
                sem,
            ).start()
        pltpu.make_async_copy(
            flat_hbm.at[pl.ds(0, rows_per_w)],
            out_hbm.at[pl.ds(base, rows_per_w)],
            sem,
        ).wait()

    return gather_kernel


def kernel(hidden_states, input_id):
    B, S, D = hidden_states.shape
    flat = hidden_states.reshape(B * S, D)
    return _build(B, S, D)(flat, input_id.astype(jnp.int32))


# speedup vs baseline: 1.5901x; 1.5901x over previous
import functools

import jax
import jax.numpy as jnp
from jax import lax
from jax.experimental import pallas as pl
from jax.experimental.pallas import tpu as pltpu
from jax.experimental.pallas import tpu_sc as plsc

_NC = 2
_NS = 16
_L = 16


@functools.lru_cache(maxsize=None)
def _build(B, S, D):
    assert B % _L == 0
    n_chunks = B // _L
    q_per_chunk = 2
    rows_per_w = _L // q_per_chunk
    n_workers = n_chunks * q_per_chunk
    mesh = plsc.VectorSubcoreMesh(
        core_axis_name="c", subcore_axis_name="s", num_cores=1)

    @functools.partial(
        pl.kernel,
        mesh=mesh,
        out_type=jax.ShapeDtypeStruct((B, D), jnp.float32),
        scratch_types=[
            pltpu.VMEM((_L,), jnp.int32),
            pltpu.VMEM((_L,), jnp.int32),
            pltpu.VMEM((rows_per_w, D), jnp.float32),
            pltpu.SemaphoreType.DMA,
        ],
    )
    def gather_kernel(flat_hbm, idx_hbm, out_hbm, ids_v, gids_v, rows_v, sem):
        wid = lax.axis_index("s")
        base = wid * rows_per_w
        pltpu.sync_copy(
            idx_hbm.at[pl.ds(base, rows_per_w)],
            ids_v.at[pl.ds(0, rows_per_w)],
        )
        lane = lax.iota(jnp.int32, _L)
        gids_v[...] = (lane + base) * S + ids_v[...]
        pltpu.async_copy(
            flat_hbm.at[gids_v.at[pl.ds(0, rows_per_w)]],
            rows_v,
            sem,
        ).wait()
        pltpu.sync_copy(rows_v, out_hbm.at[pl.ds(base, rows_per_w)])

    return gather_kernel


def kernel(hidden_states, input_id):
    B, S, D = hidden_states.shape
    flat = hidden_states.reshape(B * S, D)
    return _build(B, S, D)(flat, input_id.astype(jnp.int32))


# speedup vs baseline: 1.5982x; 1.0051x over previous
import functools

import jax
import jax.numpy as jnp
from jax import lax
from jax.experimental import pallas as pl
from jax.experimental.pallas import tpu as pltpu
from jax.experimental.pallas import tpu_sc as plsc

_NS = 16
_L = 16


@functools.lru_cache(maxsize=None)
def _build(B, S, D):
    rows_per_w = B // _NS
    assert B % _NS == 0 and rows_per_w % 8 == 0 and rows_per_w <= _L
    mesh = plsc.VectorSubcoreMesh(
        core_axis_name="c", subcore_axis_name="s", num_cores=1)

    @functools.partial(
        pl.kernel,
        mesh=mesh,
        out_type=jax.ShapeDtypeStruct((B, D), jnp.float32),
        scratch_types=[
            pltpu.VMEM((_L,), jnp.int32),
            pltpu.VMEM((_L,), jnp.int32),
            pltpu.VMEM((rows_per_w, D), jnp.float32),
            pltpu.SemaphoreType.DMA,
        ],
    )
    def gather_kernel(flat_hbm, idx_hbm, out_hbm, ids_v, gids_v, rows_v, sem):
        wid = lax.axis_index("s")
        base = wid * rows_per_w
        pltpu.sync_copy(
            idx_hbm.at[pl.ds(base, rows_per_w)],
            ids_v.at[pl.ds(0, rows_per_w)],
        )
        lane = lax.iota(jnp.int32, _L)
        gids_v[...] = (lane + base) * S + ids_v[...]
        pltpu.async_copy(
            flat_hbm.at[gids_v.at[pl.ds(0, rows_per_w)]],
            rows_v,
            sem,
        ).wait()
        pltpu.sync_copy(rows_v, out_hbm.at[pl.ds(base, rows_per_w)])

    return gather_kernel


def kernel(hidden_states, input_id):
    B, S, D = hidden_states.shape
    flat = hidden_states.reshape(B * S, D)
    return _build(B, S, D)(flat, input_id.astype(jnp.int32))
